# baseline (device time: 9505 ns/iter reference)
import jax
import jax.numpy as jnp
from jax import lax
from jax.experimental import pallas as pl
from jax.experimental.pallas import tpu as pltpu

N_GLOBAL = 1024
EPS = 1e-5


def kernel(x, gamma, beta):
    m, n = x.shape

    gamma2 = gamma.reshape(1, n)
    beta2 = beta.reshape(1, n)

    def body(x_ref, g_ref, b_ref, o_ref, stats_ref, peer_ref, send_sem, recv_sem):
        my_x = lax.axis_index("x")
        my_y = lax.axis_index("y")
        peer = (my_x, 1 - my_y)

        barrier = pltpu.get_barrier_semaphore()
        pl.semaphore_signal(
            barrier, inc=1, device_id=peer, device_id_type=pl.DeviceIdType.MESH
        )
        pl.semaphore_wait(barrier, 1)

        xv = x_ref[:, :]
        s = jnp.sum(xv, axis=1)
        ss = jnp.sum(xv * xv, axis=1)
        stats_ref[0, :] = s
        stats_ref[1, :] = ss

        rdma = pltpu.make_async_remote_copy(
            src_ref=stats_ref,
            dst_ref=peer_ref,
            send_sem=send_sem,
            recv_sem=recv_sem,
            device_id=peer,
            device_id_type=pl.DeviceIdType.MESH,
        )
        rdma.start()
        rdma.wait()

        tot_s = stats_ref[0, :] + peer_ref[0, :]
        tot_ss = stats_ref[1, :] + peer_ref[1, :]
        mean = tot_s * (1.0 / N_GLOBAL)
        var = tot_ss * (1.0 / N_GLOBAL) - mean * mean
        inv = lax.rsqrt(var + EPS)
        norm = (xv - mean[:, None]) * inv[:, None]
        o_ref[:, :] = g_ref[:, :] * norm + b_ref[:, :]

    return pl.pallas_call(
        body,
        out_shape=jax.ShapeDtypeStruct((m, n), jnp.float32),
        in_specs=[
            pl.BlockSpec(memory_space=pltpu.VMEM),
            pl.BlockSpec(memory_space=pltpu.VMEM),
            pl.BlockSpec(memory_space=pltpu.VMEM),
        ],
        out_specs=pl.BlockSpec(memory_space=pltpu.VMEM),
        scratch_shapes=[
            pltpu.VMEM((2, m), jnp.float32),
            pltpu.VMEM((2, m), jnp.float32),
            pltpu.SemaphoreType.DMA,
            pltpu.SemaphoreType.DMA,
        ],
        compiler_params=pltpu.CompilerParams(collective_id=0),
    )(x, gamma2, beta2)


# device time: 9465 ns/iter; 1.0042x vs baseline; 1.0042x over previous
import jax
import jax.numpy as jnp
from jax import lax
from jax.experimental import pallas as pl
from jax.experimental.pallas import tpu as pltpu

N_GLOBAL = 1024
EPS = 1e-5
NC = 4


def kernel(x, gamma, beta):
    m, n = x.shape
    chunk = m // NC

    gamma2 = gamma.reshape(1, n)
    beta2 = beta.reshape(1, n)

    def body(x_ref, g_ref, b_ref, o_ref, stats_ref, peer_ref, send_sems, recv_sems):
        my_x = lax.axis_index("x")
        my_y = lax.axis_index("y")
        peer = (my_x, 1 - my_y)

        barrier = pltpu.get_barrier_semaphore()
        pl.semaphore_signal(
            barrier, inc=1, device_id=peer, device_id_type=pl.DeviceIdType.MESH
        )
        pl.semaphore_wait(barrier, 1)

        g = g_ref[:, :]
        b = b_ref[:, :]

        rdmas = []
        for c in range(NC):
            xc = x_ref[c * chunk : (c + 1) * chunk, :]
            stats_ref[c, 0, :] = jnp.sum(xc, axis=1)
            stats_ref[c, 1, :] = jnp.sum(xc * xc, axis=1)
            rdma = pltpu.make_async_remote_copy(
                src_ref=stats_ref.at[c],
                dst_ref=peer_ref.at[c],
                send_sem=send_sems.at[c],
                recv_sem=recv_sems.at[c],
                device_id=peer,
                device_id_type=pl.DeviceIdType.MESH,
            )
            rdma.start()
            rdmas.append(rdma)

        for c in range(NC):
            rdmas[c].wait_recv()
            tot_s = stats_ref[c, 0, :] + peer_ref[c, 0, :]
            tot_ss = stats_ref[c, 1, :] + peer_ref[c, 1, :]
            mean = tot_s * (1.0 / N_GLOBAL)
            var = tot_ss * (1.0 / N_GLOBAL) - mean * mean
            inv = lax.rsqrt(var + EPS)
            xc = x_ref[c * chunk : (c + 1) * chunk, :]
            norm = (xc - mean[:, None]) * inv[:, None]
            o_ref[c * chunk : (c + 1) * chunk, :] = g * norm + b

        for c in range(NC):
            rdmas[c].wait_send()

    return pl.pallas_call(
        body,
        out_shape=jax.ShapeDtypeStruct((m, n), jnp.float32),
        in_specs=[
            pl.BlockSpec(memory_space=pltpu.VMEM),
            pl.BlockSpec(memory_space=pltpu.VMEM),
            pl.BlockSpec(memory_space=pltpu.VMEM),
        ],
        out_specs=pl.BlockSpec(memory_space=pltpu.VMEM),
        scratch_shapes=[
            pltpu.VMEM((NC, 2, chunk), jnp.float32),
            pltpu.VMEM((NC, 2, chunk), jnp.float32),
            pltpu.SemaphoreType.DMA((NC,)),
            pltpu.SemaphoreType.DMA((NC,)),
        ],
        compiler_params=pltpu.CompilerParams(collective_id=0),
    )(x, gamma2, beta2)
